# baseline (speedup 1.0000x reference)
import functools

import jax
import jax.numpy as jnp
from jax import lax
from jax.experimental import pallas as pl
from jax.experimental.pallas import tpu as pltpu
from jax.experimental.pallas import tpu_sc as plsc

N = 10000
E = 320000
D = 128

NC = 2
NT = 16
NW = NC * NT
EPT = E // NT
EPW = E // NW
TQ, TCH = 19968, 20480
WQ, WCH = 9984, 10496
NBINS = 10240
COLS = NBINS // NT
VPC = COLS // 16


def _nrsqrt(x):
    i = plsc.bitcast(x, jnp.int32)
    y = plsc.bitcast(jnp.int32(0x5F3759DF) - (i >> 1), jnp.float32)
    for _ in range(3):
        y = y * (1.5 - 0.5 * x * y * y)
    return y


def _sc_edge_work(edge_index):
    mesh = plsc.VectorSubcoreMesh(
        core_axis_name="c", subcore_axis_name="s", num_cores=NC,
        num_subcores=NT)

    @functools.partial(
        pl.kernel,
        out_type=(
            jax.ShapeDtypeStruct((NW, NBINS), jnp.float32),
            jax.ShapeDtypeStruct((NW, NBINS), jnp.float32),
        ),
        mesh=mesh,
        compiler_params=pltpu.CompilerParams(needs_layout_passes=False,
                                             skip_device_barrier=True),
        scratch_types=[
            pltpu.VMEM((2, WCH), jnp.int32),
            pltpu.VMEM((2, TCH), jnp.int32),
            pltpu.VMEM((NBINS,), jnp.float32),
            pltpu.VMEM((NBINS,), jnp.float32),
            pltpu.VMEM((NBINS,), jnp.float32),
            pltpu.VMEM((NT, COLS), jnp.float32),
            pltpu.VMEM((COLS,), jnp.float32),
            pltpu.VMEM_SHARED((NT, NBINS), jnp.float32),
            pltpu.VMEM_SHARED((NBINS,), jnp.float32),
            pltpu.SemaphoreType.DMA,
            pltpu.SemaphoreType.DMA,
        ],
    )
    def kern(ei_hbm, sp_hbm, od_hbm, wbuf, tbuf, ha, hb, rsqf,
             t16, buf, stB, shR, sem1, sem2):
        cid = lax.axis_index("c")
        sid = lax.axis_index("s")
        wid = sid * NC + cid
        n0 = sid * COLS
        zeros16 = jnp.zeros((16,), jnp.float32)
        ones16 = jnp.ones((16,), jnp.float32)
        iota16 = lax.iota(jnp.int32, 16)
        cnt_t = jnp.where(sid < NT - 1, TQ, TCH)
        cnt_w = jnp.where(wid < NW - 1, WQ, WCH)

        cps = pltpu.async_copy(ei_hbm.at[:, pl.ds(wid * WQ, WCH)], wbuf, sem1)
        cpd = pltpu.async_copy(ei_hbm.at[:, pl.ds(sid * TQ, TCH)], tbuf, sem2)

        @plsc.parallel_loop(0, NBINS // 16, unroll=16)
        def _(k):
            ha[pl.ds(k * 16, 16)] = zeros16
            hb[pl.ds(k * 16, 16)] = zeros16

        cps.wait()
        cpd.wait()

        @plsc.parallel_loop(0, TCH // 16, unroll=8)
        def _(i):
            o = i * 16
            m = (o + iota16) < cnt_t
            plsc.addupdate_scatter(hb, [tbuf[1, pl.ds(o, 16)]], ones16,
                                   mask=m)

        @plsc.parallel_loop(0, WCH // 16, unroll=8)
        def _(i):
            o = i * 16
            m = (o + iota16) < cnt_w
            plsc.addupdate_scatter(ha, [wbuf[0, pl.ds(o, 16)]], ones16,
                                   mask=m)

        cpha = pltpu.async_copy(ha, od_hbm.at[wid], sem1)
        pltpu.sync_copy(hb, stB.at[sid])
        plsc.subcore_barrier()
        cpha.wait()

        pltpu.sync_copy(stB.at[:, pl.ds(n0, COLS)], t16)

        @plsc.parallel_loop(0, VPC, unroll=4)
        def _(k):
            s_ = pl.ds(k * 16, 16)
            acc = t16[0, s_]
            for r in range(1, NT):
                acc = acc + t16[r, s_]
            buf[s_] = _nrsqrt(jnp.maximum(acc, 1.0))

        pltpu.sync_copy(buf, shR.at[pl.ds(n0, COLS)])
        plsc.subcore_barrier()
        cpr = pltpu.async_copy(shR, rsqf, sem1)

        @plsc.parallel_loop(0, NBINS // 16, unroll=16)
        def _(k):
            ha[pl.ds(k * 16, 16)] = zeros16

        cpr.wait()

        @plsc.parallel_loop(0, WCH // 16, unroll=8)
        def _(i):
            o = i * 16
            m = (o + iota16) < cnt_w
            v = plsc.load_gather(rsqf, [wbuf[1, pl.ds(o, 16)]], mask=m)
            plsc.addupdate_scatter(ha, [wbuf[0, pl.ds(o, 16)]], v, mask=m)

        pltpu.sync_copy(ha, sp_hbm.at[wid])

    return kern(edge_index)


CH = 1024
NCH = -(-N // CH)


def _tc_finish(s_parts, od_parts, h, W, b2, fc_W, fcb2):
    def body(sp_ref, od_ref, h_ref, W_ref, b_ref, fcW_ref, fcb_ref, o_ref,
             hb0, hb1, sem0, sem1):
        s = jnp.sum(sp_ref[...], axis=0, keepdims=True)
        od = jnp.sum(od_ref[...], axis=0, keepdims=True)
        w2 = s * lax.rsqrt(jnp.maximum(od, 1.0))

        bufs = (hb0, hb1)
        sems = (sem0, sem1)
        rows = [min(CH, N - k * CH) for k in range(NCH)]
        cps = [None] * NCH
        cps[0] = pltpu.async_copy(
            h_ref.at[pl.ds(0, rows[0])], hb0.at[pl.ds(0, rows[0])], sem0)
        v = jnp.zeros((1, D), jnp.float32)
        for k in range(NCH):
            if k + 1 < NCH:
                r = rows[k + 1]
                cps[k + 1] = pltpu.async_copy(
                    h_ref.at[pl.ds((k + 1) * CH, r)],
                    bufs[(k + 1) % 2].at[pl.ds(0, r)], sems[(k + 1) % 2])
            cps[k].wait()
            wk = lax.slice(w2, (0, k * CH), (1, (k + 1) * CH))
            v = v + lax.dot_general(wk, bufs[k % 2][...],
                                    (((1,), (0,)), ((), ())),
                                    preferred_element_type=jnp.float32,
                                    precision=lax.Precision.HIGHEST)
        hg = lax.dot_general(v, W_ref[...],
                             (((1,), (0,)), ((), ())),
                             preferred_element_type=jnp.float32,
                             precision=lax.Precision.HIGHEST)
        hg = hg * (1.0 / N) + b_ref[...]
        out = lax.dot_general(hg, fcW_ref[...],
                              (((1,), (1,)), ((), ())),
                              preferred_element_type=jnp.float32,
                              precision=lax.Precision.HIGHEST)
        o_ref[...] = out + fcb_ref[...]

    return pl.pallas_call(
        body,
        out_shape=jax.ShapeDtypeStruct((1, D), jnp.float32),
        in_specs=[
            pl.BlockSpec(memory_space=pltpu.VMEM),
            pl.BlockSpec(memory_space=pltpu.VMEM),
            pl.BlockSpec(memory_space=pl.ANY),
            pl.BlockSpec(memory_space=pltpu.VMEM),
            pl.BlockSpec(memory_space=pltpu.VMEM),
            pl.BlockSpec(memory_space=pltpu.VMEM),
            pl.BlockSpec(memory_space=pltpu.VMEM),
        ],
        scratch_shapes=[
            pltpu.VMEM((CH, D), jnp.float32),
            pltpu.VMEM((CH, D), jnp.float32),
            pltpu.SemaphoreType.DMA,
            pltpu.SemaphoreType.DMA,
        ],
        compiler_params=pltpu.CompilerParams(skip_device_barrier=True),
    )(s_parts, od_parts, h, W, b2, fc_W, fcb2)


def kernel(h, edge_index, W, b, fc_W, fc_b):
    s_parts, od_parts = _sc_edge_work(edge_index)
    return _tc_finish(s_parts, od_parts, h, W,
                      b.reshape(1, D), fc_W, fc_b.reshape(1, D))


# speedup vs baseline: 1.0797x; 1.0797x over previous
import functools

import jax
import jax.numpy as jnp
from jax import lax
from jax.experimental import pallas as pl
from jax.experimental.pallas import tpu as pltpu
from jax.experimental.pallas import tpu_sc as plsc

N = 10000
E = 320000
D = 128

NC = 2
NT = 16
NW = NC * NT
EPT = E // NT
EPW = E // NW
TQ, TCH = 19968, 20480
WQ, WCH = 9984, 10496
NBINS = 10240
COLS = NBINS // NT
VPC = COLS // 16


def _nrsqrt(x):
    i = plsc.bitcast(x, jnp.int32)
    y = plsc.bitcast(jnp.int32(0x5F3759DF) - (i >> 1), jnp.float32)
    for _ in range(3):
        y = y * (1.5 - 0.5 * x * y * y)
    return y


def _sc_edge_work(edge_index):
    mesh = plsc.VectorSubcoreMesh(
        core_axis_name="c", subcore_axis_name="s", num_cores=NC,
        num_subcores=NT)

    @functools.partial(
        pl.kernel,
        out_type=(
            jax.ShapeDtypeStruct((NW, NBINS), jnp.float32),
            jax.ShapeDtypeStruct((NW, NBINS), jnp.float32),
        ),
        mesh=mesh,
        compiler_params=pltpu.CompilerParams(needs_layout_passes=False,
                                             skip_device_barrier=True),
        scratch_types=[
            pltpu.VMEM((2, WCH), jnp.int32),
            pltpu.VMEM((2, TCH), jnp.int32),
            pltpu.VMEM((NBINS,), jnp.float32),
            pltpu.VMEM((NBINS,), jnp.float32),
            pltpu.VMEM((NBINS,), jnp.float32),
            pltpu.VMEM((NT, COLS), jnp.float32),
            pltpu.VMEM((COLS,), jnp.float32),
            pltpu.VMEM_SHARED((NT, NBINS), jnp.float32),
            pltpu.VMEM_SHARED((NBINS,), jnp.float32),
            pltpu.SemaphoreType.DMA,
            pltpu.SemaphoreType.DMA,
        ],
    )
    def kern(ei_hbm, sp_hbm, od_hbm, wbuf, tbuf, ha, hb, rsqf,
             t16, buf, stB, shR, sem1, sem2):
        cid = lax.axis_index("c")
        sid = lax.axis_index("s")
        wid = sid * NC + cid
        n0 = sid * COLS
        zeros16 = jnp.zeros((16,), jnp.float32)
        ones16 = jnp.ones((16,), jnp.float32)
        iota16 = lax.iota(jnp.int32, 16)
        cnt_t = jnp.where(sid < NT - 1, TQ, TCH)
        cnt_w = jnp.where(wid < NW - 1, WQ, WCH)

        cps = pltpu.async_copy(ei_hbm.at[:, pl.ds(wid * WQ, WCH)], wbuf, sem1)
        cpd = pltpu.async_copy(ei_hbm.at[:, pl.ds(sid * TQ, TCH)], tbuf, sem2)

        @plsc.parallel_loop(0, NBINS // 16, unroll=16)
        def _(k):
            ha[pl.ds(k * 16, 16)] = zeros16
            hb[pl.ds(k * 16, 16)] = zeros16

        cps.wait()
        cpd.wait()

        @plsc.parallel_loop(0, TCH // 16, unroll=8)
        def _(i):
            o = i * 16
            m = (o + iota16) < cnt_t
            plsc.addupdate_scatter(hb, [tbuf[1, pl.ds(o, 16)]], ones16,
                                   mask=m)

        @plsc.parallel_loop(0, WCH // 16, unroll=8)
        def _(i):
            o = i * 16
            m = (o + iota16) < cnt_w
            plsc.addupdate_scatter(ha, [wbuf[0, pl.ds(o, 16)]], ones16,
                                   mask=m)

        cpha = pltpu.async_copy(ha, od_hbm.at[wid], sem1)
        pltpu.sync_copy(hb, stB.at[sid])
        plsc.subcore_barrier()
        cpha.wait()

        pltpu.sync_copy(stB.at[:, pl.ds(n0, COLS)], t16)

        @plsc.parallel_loop(0, VPC, unroll=4)
        def _(k):
            s_ = pl.ds(k * 16, 16)
            acc = t16[0, s_]
            for r in range(1, NT):
                acc = acc + t16[r, s_]
            buf[s_] = _nrsqrt(jnp.maximum(acc, 1.0))

        pltpu.sync_copy(buf, shR.at[pl.ds(n0, COLS)])
        plsc.subcore_barrier()
        cpr = pltpu.async_copy(shR, rsqf, sem1)

        @plsc.parallel_loop(0, NBINS // 16, unroll=16)
        def _(k):
            ha[pl.ds(k * 16, 16)] = zeros16

        cpr.wait()

        @plsc.parallel_loop(0, WCH // 16, unroll=8)
        def _(i):
            o = i * 16
            m = (o + iota16) < cnt_w
            v = plsc.load_gather(rsqf, [wbuf[1, pl.ds(o, 16)]], mask=m)
            plsc.addupdate_scatter(ha, [wbuf[0, pl.ds(o, 16)]], v, mask=m)

        pltpu.sync_copy(ha, sp_hbm.at[wid])

    return kern(edge_index)


def _tc_finish(s_parts, od_parts, h, W, b2, fc_W, fcb2):
    def body(sp_ref, od_ref, h_ref, W_ref, b_ref, fcW_ref, fcb_ref, o_ref):
        s = jnp.sum(sp_ref[...], axis=0, keepdims=True)
        od = jnp.sum(od_ref[...], axis=0, keepdims=True)
        w2 = (s * lax.rsqrt(jnp.maximum(od, 1.0)))[:, :N]
        v = lax.dot_general(w2, h_ref[...],
                            (((1,), (0,)), ((), ())),
                            preferred_element_type=jnp.float32,
                            precision=lax.Precision.HIGHEST)
        hg = lax.dot_general(v, W_ref[...],
                             (((1,), (0,)), ((), ())),
                             preferred_element_type=jnp.float32,
                             precision=lax.Precision.HIGHEST)
        hg = hg * (1.0 / N) + b_ref[...]
        out = lax.dot_general(hg, fcW_ref[...],
                              (((1,), (1,)), ((), ())),
                              preferred_element_type=jnp.float32,
                              precision=lax.Precision.HIGHEST)
        o_ref[...] = out + fcb_ref[...]

    return pl.pallas_call(
        body,
        out_shape=jax.ShapeDtypeStruct((1, D), jnp.float32),
        compiler_params=pltpu.CompilerParams(skip_device_barrier=True),
    )(s_parts, od_parts, h, W, b2, fc_W, fcb2)


def kernel(h, edge_index, W, b, fc_W, fc_b):
    s_parts, od_parts = _sc_edge_work(edge_index)
    return _tc_finish(s_parts, od_parts, h, W,
                      b.reshape(1, D), fc_W, fc_b.reshape(1, D))
